# baseline (speedup 1.0000x reference)
import functools

import jax
import jax.numpy as jnp
from jax import lax
from jax.experimental import pallas as pl
from jax.experimental.pallas import tpu as pltpu
from jax.experimental.pallas import tpu_sc as plsc

N = 10000
E = 320000
F = 128
FH = F // 2
R = 16
NB = 4

NC = 2
NS = 16
EPS = E // NS
K = 400
NBATCH = EPS // K
CH = 3
NCHUNK = (NBATCH - 2) // CH
EPAD = 2 * CH * K
NPAD = 10112
ROWS_PER_TILE = NPAD // NS

TN = 2000
NT = N // TN


def _table_body(coeff_ref, feat_ref, w_ref, out_ref):
    r = pl.program_id(1)
    wr = coeff_ref[r, 0] * w_ref[0]
    for b in range(1, NB):
        wr = wr + coeff_ref[r, b] * w_ref[b]
    out_ref[...] = jnp.dot(feat_ref[...], wr, preferred_element_type=jnp.float32)


def _build_table(feat, W, coeff):
    return pl.pallas_call(
        _table_body,
        grid=(NT, R),
        in_specs=[
            pl.BlockSpec(memory_space=pltpu.SMEM),
            pl.BlockSpec((TN, F), lambda n, r: (n, 0)),
            pl.BlockSpec((NB, F, F), lambda n, r: (0, 0, 0)),
        ],
        out_specs=pl.BlockSpec((TN, F), lambda n, r: (r * NT + n, 0)),
        out_shape=jax.ShapeDtypeStruct((R * N, F), jnp.float32),
    )(coeff, feat, W)


_MESH = plsc.VectorSubcoreMesh(core_axis_name="c", subcore_axis_name="s")


@functools.partial(
    pl.kernel,
    mesh=_MESH,
    compiler_params=pltpu.CompilerParams(use_tc_tiling_on_sc=False),
    out_type=jax.ShapeDtypeStruct((NPAD, F), jnp.float32),
    scratch_types=[
        pltpu.VMEM((2 * CH * K,), jnp.int32),
        pltpu.VMEM((2 * CH * K,), jnp.int32),
        pltpu.VMEM((3 * CH, K), jnp.int32),
        [pltpu.VMEM((K,), jnp.int32)] * 3,
        [pltpu.VMEM((K, FH), jnp.float32)] * 3,
        pltpu.VMEM_SHARED((NPAD, FH), jnp.float32),
        [pltpu.SemaphoreType.DMA] * 3,
        [pltpu.SemaphoreType.DMA] * 3,
        pltpu.SemaphoreType.DMA,
    ],
)
def _edge_scatter(t_hbm, src_hbm, et_hbm, dst3_hbm, zeros_hbm, out_hbm,
                  srcv, etv, dstb, idxv, rows, acc, semG, semS, semT):
    c = lax.axis_index("c")
    s = lax.axis_index("s")

    pltpu.sync_copy(zeros_hbm.at[pl.ds(s * ROWS_PER_TILE, ROWS_PER_TILE)],
                    acc.at[pl.ds(s * ROWS_PER_TILE, ROWS_PER_TILE)])
    plsc.subcore_barrier()

    base = s * EPS

    def stage_refs(g):
        reg = lax.rem(g, 2) * (CH * K)
        off = base + g * (CH * K)
        dreg = lax.rem(g, 3) * CH
        return (
            (src_hbm.at[pl.ds(off, CH * K)], srcv.at[pl.ds(reg, CH * K)]),
            (et_hbm.at[pl.ds(off, CH * K)], etv.at[pl.ds(reg, CH * K)]),
            (dst3_hbm.at[pl.ds(s * NBATCH + g * CH, CH)],
             dstb.at[pl.ds(dreg, CH)]),
        )

    def dst_row(b):
        return lax.rem(b // CH, 3) * CH + b % CH

    def compute_and_gather(b, k, reg):
        soff = reg + (b % CH) * K
        for j in range(K // 16):
            sl = pl.ds(soff + j * 16, 16)
            idxv[k][pl.ds(j * 16, 16)] = (etv[sl] * N + srcv[sl]) * 2 + c
        pltpu.async_copy(t_hbm.at[idxv[k]], rows[k], semG[k])

    def wait_gather(k):
        pltpu.make_async_copy(t_hbm.at[idxv[k]], rows[k], semG[k]).wait()

    def scatter(b, k):
        pltpu.async_copy(rows[k], acc.at[dstb.at[dst_row(b)]], semS[k],
                         add=True)

    def drain_scatter(b, k):
        pltpu.make_async_copy(rows[k], acc.at[dstb.at[dst_row(b)]],
                              semS[k]).wait()

    for src_d, et_d, dst_d in (stage_refs(0),):
        pltpu.sync_copy(*src_d)
        pltpu.sync_copy(*et_d)
        pltpu.sync_copy(*dst_d)
    compute_and_gather(0, 0, 0)
    compute_and_gather(1, 1, 0)

    def step(g, carry):
        stage = stage_refs(g + 1)
        for src_d, et_d, dst_d in (stage,):
            pltpu.async_copy(*src_d, semT)
            pltpu.async_copy(*et_d, semT)
            pltpu.async_copy(*dst_d, semT)
        for t in range(CH):
            b = CH * g + t
            k = t
            kn = (t + 2) % CH
            reg2 = lax.rem(g + (0 if t == 0 else 1), 2) * (CH * K)
            wait_gather(k)
            scatter(b, k)

            @pl.when(b >= 1)
            def _():
                drain_scatter(b - 1, kn)

            if t == 0:
                for src_d, et_d, dst_d in (stage,):
                    pltpu.make_async_copy(*src_d, semT).wait()
                    pltpu.make_async_copy(*et_d, semT).wait()
                    pltpu.make_async_copy(*dst_d, semT).wait()
            compute_and_gather(b + 2, kn, reg2)
        return carry

    lax.fori_loop(0, NCHUNK, step, 0)

    for b in (NBATCH - 2, NBATCH - 1):
        k = b % CH
        wait_gather(k)
        scatter(b, k)
    drain_scatter(NBATCH - 3, (NBATCH - 3) % CH)
    drain_scatter(NBATCH - 2, (NBATCH - 2) % CH)
    drain_scatter(NBATCH - 1, (NBATCH - 1) % CH)

    plsc.subcore_barrier()
    pltpu.sync_copy(acc.at[pl.ds(s * ROWS_PER_TILE, ROWS_PER_TILE)],
                    out_hbm.at[pl.ds(s * ROWS_PER_TILE, ROWS_PER_TILE),
                               pl.ds(c * FH, FH)])


def _final_body(p_ref, feat_ref, lw_ref, bias_ref, out_ref):
    h = jnp.dot(feat_ref[...], lw_ref[...], preferred_element_type=jnp.float32)
    out_ref[...] = h + p_ref[...] + bias_ref[...]


def _finalize(partials, feat, loop_weight, h_bias):
    return pl.pallas_call(
        _final_body,
        grid=(NT,),
        in_specs=[
            pl.BlockSpec((TN, F), lambda n: (n, 0)),
            pl.BlockSpec((TN, F), lambda n: (n, 0)),
            pl.BlockSpec((F, F), lambda n: (0, 0)),
            pl.BlockSpec((1, F), lambda n: (0, 0)),
        ],
        out_specs=pl.BlockSpec((TN, F), lambda n: (n, 0)),
        out_shape=jax.ShapeDtypeStruct((N, F), jnp.float32),
    )(partials, feat, loop_weight, h_bias.reshape(1, F))


def kernel(feat, edge_index, etypes, W, coeff, h_bias, loop_weight):
    zeros = jnp.zeros((NPAD, FH), jnp.float32)
    src_p = jnp.pad(edge_index[0], (0, EPAD))
    et_p = jnp.pad(etypes, (0, EPAD))
    dst3 = jnp.pad(edge_index[1], (0, 8 * K)).reshape(E // K + 8, K)
    table = _build_table(feat, W, coeff)
    table64 = table.reshape(2 * R * N, FH)
    partials = _edge_scatter(table64, src_p, et_p, dst3, zeros)
    return _finalize(partials, feat, loop_weight, h_bias)


# speedup vs baseline: 1.0461x; 1.0461x over previous
import functools

import jax
import jax.numpy as jnp
from jax import lax
from jax.experimental import pallas as pl
from jax.experimental.pallas import tpu as pltpu
from jax.experimental.pallas import tpu_sc as plsc

N = 10000
E = 320000
F = 128
FH = F // 2
R = 16
NB = 4

NC = 2
NS = 16
EPS = E // NS
K = 400
NBATCH = EPS // K
CH = 3
NCHUNK = (NBATCH - 2) // CH
EPAD = 2 * CH * K
NPAD = 10112
ROWS_PER_TILE = NPAD // NS

TN = 2000
NT = N // TN


def _table_body(coeff_ref, feat_ref, w_ref, out_ref):
    r = pl.program_id(1)
    wr = coeff_ref[r, 0] * w_ref[0]
    for b in range(1, NB):
        wr = wr + coeff_ref[r, b] * w_ref[b]
    out_ref[...] = jnp.dot(feat_ref[...], wr, preferred_element_type=jnp.float32)


def _build_table(feat, W, coeff):
    return pl.pallas_call(
        _table_body,
        grid=(NT, R),
        in_specs=[
            pl.BlockSpec(memory_space=pltpu.SMEM),
            pl.BlockSpec((TN, F), lambda n, r: (n, 0)),
            pl.BlockSpec((NB, F, F), lambda n, r: (0, 0, 0)),
        ],
        out_specs=pl.BlockSpec((TN, F), lambda n, r: (r * NT + n, 0)),
        out_shape=jax.ShapeDtypeStruct((R * N, F), jnp.float32),
    )(coeff, feat, W)


_MESH = plsc.VectorSubcoreMesh(core_axis_name="c", subcore_axis_name="s")


@functools.partial(
    pl.kernel,
    mesh=_MESH,
    compiler_params=pltpu.CompilerParams(use_tc_tiling_on_sc=False),
    out_type=jax.ShapeDtypeStruct((NPAD, F), jnp.float32),
    scratch_types=[
        pltpu.VMEM((2 * CH * K,), jnp.int32),
        pltpu.VMEM((2 * CH * K,), jnp.int32),
        pltpu.VMEM((3 * CH, K), jnp.int32),
        [pltpu.VMEM((K,), jnp.int32)] * 3,
        [pltpu.VMEM((K, FH), jnp.float32)] * 3,
        pltpu.VMEM_SHARED((NPAD, FH), jnp.float32),
        [pltpu.SemaphoreType.DMA] * 3,
        [pltpu.SemaphoreType.DMA] * 3,
        pltpu.SemaphoreType.DMA,
    ],
)
def _edge_scatter(t_hbm, ei_hbm, et_hbm, zeros_hbm, out_hbm,
                  srcv, etv, dstb, idxv, rows, acc, semG, semS, semT):
    c = lax.axis_index("c")
    s = lax.axis_index("s")

    pltpu.sync_copy(zeros_hbm.at[pl.ds(s * ROWS_PER_TILE, ROWS_PER_TILE)],
                    acc.at[pl.ds(s * ROWS_PER_TILE, ROWS_PER_TILE)])
    plsc.subcore_barrier()

    base = s * EPS

    def stage_refs(g):
        reg = lax.rem(g, 2) * (CH * K)
        off = base + g * (CH * K)
        dreg = lax.rem(g, 3) * CH
        return (
            (ei_hbm.at[0, pl.ds(off, CH * K)], srcv.at[pl.ds(reg, CH * K)]),
            (et_hbm.at[pl.ds(off, CH * K)], etv.at[pl.ds(reg, CH * K)]),
        ) + tuple(
            (ei_hbm.at[1, pl.ds(off + t * K, K)], dstb.at[dreg + t])
            for t in range(CH)
        )

    def dst_row(b):
        return lax.rem(b // CH, 3) * CH + b % CH

    def compute_and_gather(b, k, reg):
        soff = reg + (b % CH) * K
        for j in range(K // 16):
            sl = pl.ds(soff + j * 16, 16)
            idxv[k][pl.ds(j * 16, 16)] = (etv[sl] * N + srcv[sl]) * 2 + c
        pltpu.async_copy(t_hbm.at[idxv[k]], rows[k], semG[k])

    def wait_gather(k):
        pltpu.make_async_copy(t_hbm.at[idxv[k]], rows[k], semG[k]).wait()

    def scatter(b, k):
        pltpu.async_copy(rows[k], acc.at[dstb.at[dst_row(b)]], semS[k],
                         add=True)

    def drain_scatter(b, k):
        pltpu.make_async_copy(rows[k], acc.at[dstb.at[dst_row(b)]],
                              semS[k]).wait()

    for pair in stage_refs(0):
        pltpu.sync_copy(*pair)
    compute_and_gather(0, 0, 0)
    compute_and_gather(1, 1, 0)

    def step(g, carry):
        stage = stage_refs(g + 1)
        for pair in stage:
            pltpu.async_copy(*pair, semT)
        for t in range(CH):
            b = CH * g + t
            k = t
            kn = (t + 2) % CH
            reg2 = lax.rem(g + (0 if t == 0 else 1), 2) * (CH * K)
            wait_gather(k)
            scatter(b, k)

            @pl.when(b >= 1)
            def _():
                drain_scatter(b - 1, kn)

            if t == 0:
                for pair in stage:
                    pltpu.make_async_copy(*pair, semT).wait()
            compute_and_gather(b + 2, kn, reg2)
        return carry

    lax.fori_loop(0, NCHUNK, step, 0)

    for b in (NBATCH - 2, NBATCH - 1):
        k = b % CH
        wait_gather(k)
        scatter(b, k)
    drain_scatter(NBATCH - 3, (NBATCH - 3) % CH)
    drain_scatter(NBATCH - 2, (NBATCH - 2) % CH)
    drain_scatter(NBATCH - 1, (NBATCH - 1) % CH)

    plsc.subcore_barrier()
    pltpu.sync_copy(acc.at[pl.ds(s * ROWS_PER_TILE, ROWS_PER_TILE)],
                    out_hbm.at[pl.ds(s * ROWS_PER_TILE, ROWS_PER_TILE),
                               pl.ds(c * FH, FH)])


def _final_body(p_ref, feat_ref, lw_ref, bias_ref, out_ref):
    h = jnp.dot(feat_ref[...], lw_ref[...], preferred_element_type=jnp.float32)
    out_ref[...] = h + p_ref[...] + bias_ref[...]


def _finalize(partials, feat, loop_weight, h_bias):
    return pl.pallas_call(
        _final_body,
        grid=(NT,),
        in_specs=[
            pl.BlockSpec((TN, F), lambda n: (n, 0)),
            pl.BlockSpec((TN, F), lambda n: (n, 0)),
            pl.BlockSpec((F, F), lambda n: (0, 0)),
            pl.BlockSpec((1, F), lambda n: (0, 0)),
        ],
        out_specs=pl.BlockSpec((TN, F), lambda n: (n, 0)),
        out_shape=jax.ShapeDtypeStruct((N, F), jnp.float32),
    )(partials, feat, loop_weight, h_bias.reshape(1, F))


def kernel(feat, edge_index, etypes, W, coeff, h_bias, loop_weight):
    zeros = jnp.zeros((NPAD, FH), jnp.float32)
    ei_p = jnp.pad(edge_index, ((0, 0), (0, EPAD)))
    et_p = jnp.pad(etypes, (0, EPAD))
    table = _build_table(feat, W, coeff)
    table64 = table.reshape(2 * R * N, FH)
    partials = _edge_scatter(table64, ei_p, et_p, zeros)
    return _finalize(partials, feat, loop_weight, h_bias)


# speedup vs baseline: 1.0781x; 1.0306x over previous
import functools

import jax
import jax.numpy as jnp
from jax import lax
from jax.experimental import pallas as pl
from jax.experimental.pallas import tpu as pltpu
from jax.experimental.pallas import tpu_sc as plsc

N = 10000
E = 320000
F = 128
FH = F // 2
R = 16
NB = 4

NC = 2
NS = 16
EPS = E // NS
K = 400
NBATCH = EPS // K
CH = 3
NCHUNK = (NBATCH - 2) // CH
EPAD = 2 * CH * K
NPAD = 10112
ROWS_PER_TILE = NPAD // NS

TN = 2000
NT = N // TN


def _table_body(coeff_ref, feat_ref, w_ref, lw_ref, bias_ref,
                out_ref, hself_ref):
    r = pl.program_id(1)
    wr = coeff_ref[r, 0] * w_ref[0]
    for b in range(1, NB):
        wr = wr + coeff_ref[r, b] * w_ref[b]
    out_ref[...] = jnp.dot(feat_ref[...], wr, preferred_element_type=jnp.float32)

    @pl.when(r == 0)
    def _():
        hself_ref[...] = (
            jnp.dot(feat_ref[...], lw_ref[...],
                    preferred_element_type=jnp.float32) + bias_ref[...])


def _build_table(feat, W, coeff, loop_weight, h_bias):
    return pl.pallas_call(
        _table_body,
        grid=(NT, R),
        in_specs=[
            pl.BlockSpec(memory_space=pltpu.SMEM),
            pl.BlockSpec((TN, F), lambda n, r: (n, 0)),
            pl.BlockSpec((NB, F, F), lambda n, r: (0, 0, 0)),
            pl.BlockSpec((F, F), lambda n, r: (0, 0)),
            pl.BlockSpec((1, F), lambda n, r: (0, 0)),
        ],
        out_specs=[
            pl.BlockSpec((TN, F), lambda n, r: (r * NT + n, 0)),
            pl.BlockSpec((TN, F), lambda n, r: (n, 0)),
        ],
        out_shape=[
            jax.ShapeDtypeStruct((R * N, F), jnp.float32),
            jax.ShapeDtypeStruct((N, F), jnp.float32),
        ],
    )(coeff, feat, W, loop_weight, h_bias.reshape(1, F))


_MESH = plsc.VectorSubcoreMesh(core_axis_name="c", subcore_axis_name="s")


@functools.partial(
    pl.kernel,
    mesh=_MESH,
    compiler_params=pltpu.CompilerParams(use_tc_tiling_on_sc=False),
    out_type=jax.ShapeDtypeStruct((N, F), jnp.float32),
    scratch_types=[
        pltpu.VMEM((2 * CH * K,), jnp.int32),
        pltpu.VMEM((2 * CH * K,), jnp.int32),
        pltpu.VMEM((3 * CH, K), jnp.int32),
        [pltpu.VMEM((K,), jnp.int32)] * 3,
        [pltpu.VMEM((K, FH), jnp.float32)] * 3,
        pltpu.VMEM_SHARED((NPAD, FH), jnp.float32),
        [pltpu.SemaphoreType.DMA] * 3,
        [pltpu.SemaphoreType.DMA] * 3,
        pltpu.SemaphoreType.DMA,
    ],
)
def _edge_scatter(t_hbm, ei_hbm, et_hbm, hself_hbm, out_hbm,
                  srcv, etv, dstb, idxv, rows, acc, semG, semS, semT):
    c = lax.axis_index("c")
    s = lax.axis_index("s")

    col = pl.ds(c * FH, FH)

    @pl.when(s < NS - 1)
    def _():
        rs = pl.ds(s * ROWS_PER_TILE, ROWS_PER_TILE)
        pltpu.sync_copy(hself_hbm.at[rs, col], acc.at[rs])

    @pl.when(s == NS - 1)
    def _():
        rs = pl.ds((NS - 1) * ROWS_PER_TILE, N - (NS - 1) * ROWS_PER_TILE)
        pltpu.sync_copy(hself_hbm.at[rs, col], acc.at[rs])

    plsc.subcore_barrier()

    base = s * EPS

    def stage_refs(g):
        reg = lax.rem(g, 2) * (CH * K)
        off = base + g * (CH * K)
        dreg = lax.rem(g, 3) * CH
        return (
            (ei_hbm.at[0, pl.ds(off, CH * K)], srcv.at[pl.ds(reg, CH * K)]),
            (et_hbm.at[pl.ds(off, CH * K)], etv.at[pl.ds(reg, CH * K)]),
        ) + tuple(
            (ei_hbm.at[1, pl.ds(off + t * K, K)], dstb.at[dreg + t])
            for t in range(CH)
        )

    def dst_row(b):
        return lax.rem(b // CH, 3) * CH + b % CH

    def compute_and_gather(b, k, reg):
        soff = reg + (b % CH) * K
        for j in range(K // 16):
            sl = pl.ds(soff + j * 16, 16)
            idxv[k][pl.ds(j * 16, 16)] = (etv[sl] * N + srcv[sl]) * 2 + c
        pltpu.async_copy(t_hbm.at[idxv[k]], rows[k], semG[k])

    def wait_gather(k):
        pltpu.make_async_copy(t_hbm.at[idxv[k]], rows[k], semG[k]).wait()

    def scatter(b, k):
        pltpu.async_copy(rows[k], acc.at[dstb.at[dst_row(b)]], semS[k],
                         add=True)

    def drain_scatter(b, k):
        pltpu.make_async_copy(rows[k], acc.at[dstb.at[dst_row(b)]],
                              semS[k]).wait()

    for pair in stage_refs(0):
        pltpu.sync_copy(*pair)
    compute_and_gather(0, 0, 0)
    compute_and_gather(1, 1, 0)

    def step(g, carry):
        stage = stage_refs(g + 1)
        for pair in stage:
            pltpu.async_copy(*pair, semT)
        for t in range(CH):
            b = CH * g + t
            k = t
            kn = (t + 2) % CH
            reg2 = lax.rem(g + (0 if t == 0 else 1), 2) * (CH * K)
            wait_gather(k)
            scatter(b, k)

            @pl.when(b >= 1)
            def _():
                drain_scatter(b - 1, kn)

            if t == 0:
                for pair in stage:
                    pltpu.make_async_copy(*pair, semT).wait()
            compute_and_gather(b + 2, kn, reg2)
        return carry

    lax.fori_loop(0, NCHUNK, step, 0)

    for b in (NBATCH - 2, NBATCH - 1):
        k = b % CH
        wait_gather(k)
        scatter(b, k)
    drain_scatter(NBATCH - 3, (NBATCH - 3) % CH)
    drain_scatter(NBATCH - 2, (NBATCH - 2) % CH)
    drain_scatter(NBATCH - 1, (NBATCH - 1) % CH)

    plsc.subcore_barrier()

    @pl.when(s < NS - 1)
    def _():
        rs = pl.ds(s * ROWS_PER_TILE, ROWS_PER_TILE)
        pltpu.sync_copy(acc.at[rs], out_hbm.at[rs, col])

    @pl.when(s == NS - 1)
    def _():
        rs = pl.ds((NS - 1) * ROWS_PER_TILE, N - (NS - 1) * ROWS_PER_TILE)
        pltpu.sync_copy(acc.at[rs], out_hbm.at[rs, col])


def kernel(feat, edge_index, etypes, W, coeff, h_bias, loop_weight):
    ei_p = jnp.pad(edge_index, ((0, 0), (0, EPAD)))
    et_p = jnp.pad(etypes, (0, EPAD))
    table, hself = _build_table(feat, W, coeff, loop_weight, h_bias)
    table64 = table.reshape(2 * R * N, FH)
    return _edge_scatter(table64, ei_p, et_p, hself)


# speedup vs baseline: 1.1042x; 1.0242x over previous
import functools

import jax
import jax.numpy as jnp
from jax import lax
from jax.experimental import pallas as pl
from jax.experimental.pallas import tpu as pltpu
from jax.experimental.pallas import tpu_sc as plsc

N = 10000
E = 320000
F = 128
FH = F // 2
R = 16
NB = 4

NC = 2
NS = 16
EPS = E // NS
K = 400
NBATCH = EPS // K
CH = 3
NCHUNK = (NBATCH - 2) // CH
EPAD = 2 * CH * K
NPAD = 10112
ROWS_PER_TILE = NPAD // NS

TN = 2000
NT = N // TN


def _table_body(coeff_ref, feat_ref, w_ref, lw_ref, bias_ref,
                out_ref, hself_ref):
    r = pl.program_id(1)
    wr = coeff_ref[r, 0] * w_ref[0]
    for b in range(1, NB):
        wr = wr + coeff_ref[r, b] * w_ref[b]
    out_ref[...] = jnp.dot(feat_ref[...], wr, preferred_element_type=jnp.float32)

    @pl.when(r == 0)
    def _():
        hself_ref[...] = (
            jnp.dot(feat_ref[...], lw_ref[...],
                    preferred_element_type=jnp.float32) + bias_ref[...])


def _build_table(feat, W, coeff, loop_weight, h_bias):
    return pl.pallas_call(
        _table_body,
        grid=(NT, R),
        in_specs=[
            pl.BlockSpec(memory_space=pltpu.SMEM),
            pl.BlockSpec((TN, F), lambda n, r: (n, 0)),
            pl.BlockSpec((NB, F, F), lambda n, r: (0, 0, 0)),
            pl.BlockSpec((F, F), lambda n, r: (0, 0)),
            pl.BlockSpec((1, F), lambda n, r: (0, 0)),
        ],
        out_specs=[
            pl.BlockSpec((TN, F), lambda n, r: (r * NT + n, 0)),
            pl.BlockSpec((TN, F), lambda n, r: (n, 0)),
        ],
        out_shape=[
            jax.ShapeDtypeStruct((R * N, F), jnp.float32),
            jax.ShapeDtypeStruct((N, F), jnp.float32),
        ],
    )(coeff, feat, W, loop_weight, h_bias.reshape(1, F))


_MESH = plsc.VectorSubcoreMesh(core_axis_name="c", subcore_axis_name="s")


@functools.partial(
    pl.kernel,
    mesh=_MESH,
    compiler_params=pltpu.CompilerParams(use_tc_tiling_on_sc=False),
    out_type=jax.ShapeDtypeStruct((N, F), jnp.float32),
    scratch_types=[
        pltpu.VMEM((2 * CH * K,), jnp.int32),
        pltpu.VMEM((2 * CH * K,), jnp.int32),
        pltpu.VMEM((3 * CH, K), jnp.int32),
        [pltpu.VMEM((K,), jnp.int32)] * 3,
        [pltpu.VMEM((K, FH), jnp.float32)] * 3,
        pltpu.VMEM_SHARED((NPAD, FH), jnp.float32),
        [pltpu.SemaphoreType.DMA] * 3,
        [pltpu.SemaphoreType.DMA] * 3,
        pltpu.SemaphoreType.DMA,
    ],
)
def _edge_scatter(t_hbm, ei_hbm, et_hbm, hself_hbm, out_hbm,
                  srcv, etv, dstb, idxv, rows, acc, semG, semS, semT):
    c = lax.axis_index("c")
    s = lax.axis_index("s")

    col = pl.ds(c * FH, FH)

    @pl.when(s < NS - 1)
    def _():
        rs = pl.ds(s * ROWS_PER_TILE, ROWS_PER_TILE)
        pltpu.sync_copy(hself_hbm.at[rs, col], acc.at[rs])

    @pl.when(s == NS - 1)
    def _():
        rs = pl.ds((NS - 1) * ROWS_PER_TILE, N - (NS - 1) * ROWS_PER_TILE)
        pltpu.sync_copy(hself_hbm.at[rs, col], acc.at[rs])

    plsc.subcore_barrier()

    base = s * EPS

    def stage_refs(g):
        reg = lax.rem(g, 2) * (CH * K)
        off = base + g * (CH * K)
        dreg = lax.rem(g, 3) * CH
        return (
            (ei_hbm.at[0, pl.ds(off, CH * K)], srcv.at[pl.ds(reg, CH * K)]),
            (et_hbm.at[pl.ds(off, CH * K)], etv.at[pl.ds(reg, CH * K)]),
        ) + tuple(
            (ei_hbm.at[1, pl.ds(off + t * K, K)], dstb.at[dreg + t])
            for t in range(CH)
        )

    def dst_row(b):
        return lax.rem(b // CH, 3) * CH + b % CH

    def compute_idx(b, k, reg):
        soff = reg + (b % CH) * K
        for j in range(K // 16):
            sl = pl.ds(soff + j * 16, 16)
            idxv[k][pl.ds(j * 16, 16)] = (etv[sl] * N + srcv[sl]) * 2 + c

    def issue_gather(k):
        pltpu.async_copy(t_hbm.at[idxv[k]], rows[k], semG[k])

    def compute_and_gather(b, k, reg):
        compute_idx(b, k, reg)
        issue_gather(k)

    def wait_gather(k):
        pltpu.make_async_copy(t_hbm.at[idxv[k]], rows[k], semG[k]).wait()

    def scatter(b, k):
        pltpu.async_copy(rows[k], acc.at[dstb.at[dst_row(b)]], semS[k],
                         add=True)

    def drain_scatter(b, k):
        pltpu.make_async_copy(rows[k], acc.at[dstb.at[dst_row(b)]],
                              semS[k]).wait()

    for pair in stage_refs(0):
        pltpu.sync_copy(*pair)
    compute_and_gather(0, 0, 0)
    compute_and_gather(1, 1, 0)

    def step(g, carry):
        stage = stage_refs(g + 1)
        for pair in stage:
            pltpu.async_copy(*pair, semT)
        for t in range(CH):
            b = CH * g + t
            k = t
            kn = (t + 2) % CH
            reg2 = lax.rem(g + (0 if t == 0 else 1), 2) * (CH * K)
            wait_gather(k)
            scatter(b, k)
            compute_idx(b + 2, kn, reg2)

            @pl.when(b >= 1)
            def _():
                drain_scatter(b - 1, kn)

            issue_gather(kn)
            if t == 0:
                for pair in stage:
                    pltpu.make_async_copy(*pair, semT).wait()
        return carry

    lax.fori_loop(0, NCHUNK, step, 0)

    for b in (NBATCH - 2, NBATCH - 1):
        k = b % CH
        wait_gather(k)
        scatter(b, k)
    drain_scatter(NBATCH - 3, (NBATCH - 3) % CH)
    drain_scatter(NBATCH - 2, (NBATCH - 2) % CH)
    drain_scatter(NBATCH - 1, (NBATCH - 1) % CH)

    plsc.subcore_barrier()

    @pl.when(s < NS - 1)
    def _():
        rs = pl.ds(s * ROWS_PER_TILE, ROWS_PER_TILE)
        pltpu.sync_copy(acc.at[rs], out_hbm.at[rs, col])

    @pl.when(s == NS - 1)
    def _():
        rs = pl.ds((NS - 1) * ROWS_PER_TILE, N - (NS - 1) * ROWS_PER_TILE)
        pltpu.sync_copy(acc.at[rs], out_hbm.at[rs, col])


def kernel(feat, edge_index, etypes, W, coeff, h_bias, loop_weight):
    ei_p = jnp.pad(edge_index, ((0, 0), (0, EPAD)))
    et_p = jnp.pad(etypes, (0, EPAD))
    table, hself = _build_table(feat, W, coeff, loop_weight, h_bias)
    table64 = table.reshape(2 * R * N, FH)
    return _edge_scatter(table64, ei_p, et_p, hself)
